# baseline (speedup 1.0000x reference)
import functools
import jax
import jax.numpy as jnp
from jax import lax
from jax.experimental import pallas as pl
from jax.experimental.pallas import tpu as pltpu
from jax.experimental.pallas import tpu_sc as plsc

NC = 2
NS = 16
L = 16

ROWS = 128
COLS = 32768
ROWS_PER_TILE = ROWS // (NC * NS)
UNROLL = 8
I32_MAX = 2147483647


def _argmax_body(x_hbm, out_hbm, buf_v, res_v):
    c = lax.axis_index("c")
    s = lax.axis_index("s")
    wid = c * NS + s
    lane = lax.iota(jnp.int32, L)
    n_steps = COLS // (UNROLL * L)

    acc = jnp.zeros((L,), jnp.int32)
    for r in range(ROWS_PER_TILE):
        row = wid * ROWS_PER_TILE + r
        pltpu.sync_copy(x_hbm.at[row], buf_v)

        def body(i, carry):
            best, bidx = carry
            base = i * (UNROLL * L)
            for j in range(UNROLL):
                off = base + j * L
                v = buf_v[pl.ds(off, L)]
                m = v > best
                best = jnp.maximum(best, v)
                bidx = jnp.where(m, lane + off, bidx)
            return best, bidx

        best0 = jnp.full((L,), -jnp.inf, jnp.float32)
        bidx0 = jnp.zeros((L,), jnp.int32)
        best, bidx = lax.fori_loop(0, n_steps, body, (best0, bidx0))

        mx = jnp.max(best)
        cand = jnp.where(best == mx, bidx, jnp.full((L,), I32_MAX, jnp.int32))
        acc = jnp.where(lane == r, jnp.min(cand), acc)

    res_v[...] = acc
    pltpu.sync_copy(res_v, out_hbm.at[pl.ds(wid * L, L)])


@jax.jit
def _argmax_sc(x):
    mesh = plsc.VectorSubcoreMesh(core_axis_name="c", subcore_axis_name="s")
    out = pl.kernel(
        _argmax_body,
        out_type=jax.ShapeDtypeStruct((NC * NS * L,), jnp.int32),
        mesh=mesh,
        scratch_types=[
            pltpu.VMEM((COLS,), jnp.float32),
            pltpu.VMEM((L,), jnp.int32),
        ],
        compiler_params=pltpu.CompilerParams(needs_layout_passes=False),
    )(x)
    return out.reshape(NC * NS, L)[:, :ROWS_PER_TILE].reshape(ROWS, 1)


def kernel(inputs):
    return _argmax_sc(inputs)


# speedup vs baseline: 1.1951x; 1.1951x over previous
import functools
import jax
import jax.numpy as jnp
from jax import lax
from jax.experimental import pallas as pl
from jax.experimental.pallas import tpu as pltpu
from jax.experimental.pallas import tpu_sc as plsc

NC = 2
NS = 16
L = 16

ROWS = 128
COLS = 32768
ROWS_PER_TILE = ROWS // (NC * NS)
UNROLL = 8
I32_MAX = 2147483647


def _argmax_body(x_hbm, out_hbm, buf0, buf1, res_v, sem0, sem1):
    c = lax.axis_index("c")
    s = lax.axis_index("s")
    wid = c * NS + s
    lane = lax.iota(jnp.int32, L)
    n_steps = COLS // (UNROLL * L)
    bufs = [buf0, buf1]
    sems = [sem0, sem1]

    acc = jnp.zeros((L,), jnp.int32)
    copies = [pltpu.async_copy(x_hbm.at[wid * ROWS_PER_TILE], buf0, sem0)]
    for r in range(ROWS_PER_TILE):
        if r + 1 < ROWS_PER_TILE:
            copies.append(pltpu.async_copy(
                x_hbm.at[wid * ROWS_PER_TILE + r + 1],
                bufs[(r + 1) % 2], sems[(r + 1) % 2]))
        copies[r].wait()
        buf_v = bufs[r % 2]

        def body(i, carry):
            best, bidx = carry
            for j in range(UNROLL):
                cid = i * UNROLL + j
                v = buf_v[pl.ds(cid * L, L)]
                m = v > best
                best = jnp.maximum(best, v)
                bidx = jnp.where(m, cid, bidx)
            return best, bidx

        best0 = jnp.full((L,), -jnp.inf, jnp.float32)
        bidx0 = jnp.zeros((L,), jnp.int32)
        best, bidx = lax.fori_loop(0, n_steps, body, (best0, bidx0))

        elem = (bidx << 4) | lane
        mx = jnp.max(best)
        cand = jnp.where(best == mx, elem, jnp.full((L,), I32_MAX, jnp.int32))
        acc = jnp.where(lane == r, jnp.min(cand), acc)

    res_v[...] = acc
    pltpu.sync_copy(res_v, out_hbm.at[pl.ds(wid * L, L)])


@jax.jit
def _argmax_sc(x):
    mesh = plsc.VectorSubcoreMesh(core_axis_name="c", subcore_axis_name="s")
    out = pl.kernel(
        _argmax_body,
        out_type=jax.ShapeDtypeStruct((NC * NS * L,), jnp.int32),
        mesh=mesh,
        scratch_types=[
            pltpu.VMEM((COLS,), jnp.float32),
            pltpu.VMEM((COLS,), jnp.float32),
            pltpu.VMEM((L,), jnp.int32),
            pltpu.SemaphoreType.DMA,
            pltpu.SemaphoreType.DMA,
        ],
        compiler_params=pltpu.CompilerParams(needs_layout_passes=False),
    )(x)
    return out.reshape(NC * NS, L)[:, :ROWS_PER_TILE].reshape(ROWS, 1)


def kernel(inputs):
    return _argmax_sc(inputs)


# speedup vs baseline: 1.2439x; 1.0408x over previous
import functools
import jax
import jax.numpy as jnp
from jax import lax
from jax.experimental import pallas as pl
from jax.experimental.pallas import tpu as pltpu
from jax.experimental.pallas import tpu_sc as plsc

NC = 2
NS = 16
L = 16

ROWS = 128
COLS = 32768
ROWS_PER_TILE = ROWS // (NC * NS)
UNROLL = 8
I32_MAX = 2147483647


def _argmax_body(x_hbm, out_hbm, buf_v, res_v, sem):
    c = lax.axis_index("c")
    s = lax.axis_index("s")
    wid = c * NS + s
    lane = lax.iota(jnp.int32, L)
    n_steps = COLS // (UNROLL * L)
    row0 = wid * ROWS_PER_TILE

    def copy_row(r, parity):
        return pltpu.make_async_copy(
            x_hbm.at[row0 + r], buf_v.at[pl.ds(parity * COLS, COLS)], sem)

    copy_row(0, 0).start()

    def row_body(r, acc):
        parity = lax.rem(r, 2)

        @pl.when(r + 1 < ROWS_PER_TILE)
        def _():
            copy_row(r + 1, 1 - parity).start()

        copy_row(r, parity).wait()
        base = parity * COLS

        def body(i, carry):
            best, bidx = carry
            for j in range(UNROLL):
                cid = i * UNROLL + j
                v = buf_v[pl.ds(base + cid * L, L)]
                m = v > best
                best = jnp.maximum(best, v)
                bidx = jnp.where(m, cid, bidx)
            return best, bidx

        best0 = jnp.full((L,), -jnp.inf, jnp.float32)
        bidx0 = jnp.zeros((L,), jnp.int32)
        best, bidx = lax.fori_loop(0, n_steps, body, (best0, bidx0))

        elem = (bidx << 4) | lane
        mx = jnp.max(best)
        cand = jnp.where(best == mx, elem, jnp.full((L,), I32_MAX, jnp.int32))
        return jnp.where(lane == r, jnp.min(cand), acc)

    acc = lax.fori_loop(0, ROWS_PER_TILE, row_body, jnp.zeros((L,), jnp.int32))
    res_v[...] = acc
    pltpu.sync_copy(res_v, out_hbm.at[pl.ds(wid * L, L)])


@jax.jit
def _argmax_sc(x):
    mesh = plsc.VectorSubcoreMesh(core_axis_name="c", subcore_axis_name="s")
    out = pl.kernel(
        _argmax_body,
        out_type=jax.ShapeDtypeStruct((NC * NS * L,), jnp.int32),
        mesh=mesh,
        scratch_types=[
            pltpu.VMEM((2 * COLS,), jnp.float32),
            pltpu.VMEM((L,), jnp.int32),
            pltpu.SemaphoreType.DMA,
        ],
        compiler_params=pltpu.CompilerParams(needs_layout_passes=False),
    )(x)
    return out.reshape(NC * NS, L)[:, :ROWS_PER_TILE].reshape(ROWS, 1)


def kernel(inputs):
    return _argmax_sc(inputs)


# speedup vs baseline: 2.1391x; 1.7196x over previous
import jax
import jax.numpy as jnp
from jax import lax
from jax.experimental import pallas as pl
from jax.experimental.pallas import tpu as pltpu

ROWS = 128
COLS = 32768
RB = 8
LANES = 128
NCH = COLS // LANES
I32_MAX = 2147483647


def _argmax_block(x_ref, o_ref):
    best = x_ref[:, 0:LANES]
    bidx = jnp.zeros((RB, LANES), jnp.int32)
    for k in range(1, NCH):
        v = x_ref[:, k * LANES:(k + 1) * LANES]
        m = v > best
        best = jnp.where(m, v, best)
        bidx = jnp.where(m, k, bidx)

    lanei = lax.broadcasted_iota(jnp.int32, (RB, LANES), 1)
    elem = (bidx << 7) | lanei
    mx = jnp.max(best, axis=1, keepdims=True)
    cand = jnp.where(best == mx, elem, I32_MAX)
    o_ref[...] = jnp.min(cand, axis=1, keepdims=True)


@jax.jit
def _argmax_tc(x):
    return pl.pallas_call(
        _argmax_block,
        grid=(ROWS // RB,),
        in_specs=[pl.BlockSpec((RB, COLS), lambda i: (i, 0))],
        out_specs=pl.BlockSpec((RB, 1), lambda i: (i, 0)),
        out_shape=jax.ShapeDtypeStruct((ROWS, 1), jnp.int32),
    )(x)


def kernel(inputs):
    return _argmax_tc(inputs)


# speedup vs baseline: 2.2971x; 1.0739x over previous
import jax
import jax.numpy as jnp
from jax import lax
from jax.experimental import pallas as pl
from jax.experimental.pallas import tpu as pltpu

ROWS = 128
COLS = 32768
RB = 8
LANES = 128
NCH = COLS // LANES
NACC = 8
I32_MAX = 2147483647


def _argmax_block(x_ref, o_ref):
    bests = []
    bidxs = []
    for a in range(NACC):
        bests.append(x_ref[:, a * LANES:(a + 1) * LANES])
        bidxs.append(jnp.full((RB, LANES), a, jnp.int32))
    for k in range(NACC, NCH):
        a = k % NACC
        v = x_ref[:, k * LANES:(k + 1) * LANES]
        m = v > bests[a]
        bests[a] = jnp.where(m, v, bests[a])
        bidxs[a] = jnp.where(m, k, bidxs[a])

    n = NACC
    while n > 1:
        for a in range(n // 2):
            vl, il = bests[2 * a], bidxs[2 * a]
            vr, ir = bests[2 * a + 1], bidxs[2 * a + 1]
            m = (vr > vl) | ((vr == vl) & (ir < il))
            bests[a] = jnp.where(m, vr, vl)
            bidxs[a] = jnp.where(m, ir, il)
        n //= 2
    best, bidx = bests[0], bidxs[0]

    lanei = lax.broadcasted_iota(jnp.int32, (RB, LANES), 1)
    elem = (bidx << 7) | lanei
    mx = jnp.max(best, axis=1, keepdims=True)
    cand = jnp.where(best == mx, elem, I32_MAX)
    o_ref[...] = jnp.min(cand, axis=1, keepdims=True)


@jax.jit
def _argmax_tc(x):
    return pl.pallas_call(
        _argmax_block,
        grid=(ROWS // RB,),
        in_specs=[pl.BlockSpec((RB, COLS), lambda i: (i, 0))],
        out_specs=pl.BlockSpec((RB, 1), lambda i: (i, 0)),
        out_shape=jax.ShapeDtypeStruct((ROWS, 1), jnp.int32),
    )(x)


def kernel(inputs):
    return _argmax_tc(inputs)


# speedup vs baseline: 2.2989x; 1.0007x over previous
import jax
import jax.numpy as jnp
from jax import lax
from jax.experimental import pallas as pl
from jax.experimental.pallas import tpu as pltpu

ROWS = 128
COLS = 32768
RB = 8
LANES = 128
NCH = COLS // LANES
NACC = 16
I32_MAX = 2147483647


def _argmax_block(x_ref, o_ref):
    bests = []
    bidxs = []
    for a in range(NACC):
        bests.append(x_ref[:, a * LANES:(a + 1) * LANES])
        bidxs.append(jnp.full((RB, LANES), a, jnp.int32))
    for k in range(NACC, NCH):
        a = k % NACC
        v = x_ref[:, k * LANES:(k + 1) * LANES]
        m = v > bests[a]
        bests[a] = jnp.where(m, v, bests[a])
        bidxs[a] = jnp.where(m, k, bidxs[a])

    n = NACC
    while n > 1:
        for a in range(n // 2):
            vl, il = bests[2 * a], bidxs[2 * a]
            vr, ir = bests[2 * a + 1], bidxs[2 * a + 1]
            m = (vr > vl) | ((vr == vl) & (ir < il))
            bests[a] = jnp.where(m, vr, vl)
            bidxs[a] = jnp.where(m, ir, il)
        n //= 2
    best, bidx = bests[0], bidxs[0]

    lanei = lax.broadcasted_iota(jnp.int32, (RB, LANES), 1)
    elem = (bidx << 7) | lanei
    mx = jnp.max(best, axis=1, keepdims=True)
    cand = jnp.where(best == mx, elem, I32_MAX)
    o_ref[...] = jnp.min(cand, axis=1, keepdims=True)


@jax.jit
def _argmax_tc(x):
    return pl.pallas_call(
        _argmax_block,
        grid=(ROWS // RB,),
        in_specs=[pl.BlockSpec((RB, COLS), lambda i: (i, 0))],
        out_specs=pl.BlockSpec((RB, 1), lambda i: (i, 0)),
        out_shape=jax.ShapeDtypeStruct((ROWS, 1), jnp.int32),
        compiler_params=pltpu.CompilerParams(
            dimension_semantics=("parallel",)),
    )(x)


def kernel(inputs):
    return _argmax_tc(inputs)


# speedup vs baseline: 3.8510x; 1.6752x over previous
import jax
import jax.numpy as jnp
from jax import lax
from jax.experimental import pallas as pl
from jax.experimental.pallas import tpu as pltpu

ROWS = 128
COLS = 32768
RB = 8
NBLK = ROWS // RB
LANES = 128
NCH = COLS // LANES
NACC = 8
NBUF = 4
I32_MAX = 2147483647


def _blk_copy(x_any, big, sems, i, slot):
    return pltpu.make_async_copy(
        x_any.at[pl.ds(i * RB, RB)],
        big.at[pl.ds(slot * RB, RB)],
        sems.at[slot])


def _argmax_body(x_any, o_ref, big, sems):
    for i in range(NBUF):
        _blk_copy(x_any, big, sems, i, i).start()

    def step(i, _):
        slot = lax.rem(i, NBUF)
        _blk_copy(x_any, big, sems, i, slot).wait()
        base = slot * RB

        bests = []
        bidxs = []
        for a in range(NACC):
            bests.append(big[pl.ds(base, RB), a * LANES:(a + 1) * LANES])
            bidxs.append(jnp.full((RB, LANES), a, jnp.int32))
        for k in range(NACC, NCH):
            a = k % NACC
            v = big[pl.ds(base, RB), k * LANES:(k + 1) * LANES]
            m = v > bests[a]
            bests[a] = jnp.where(m, v, bests[a])
            bidxs[a] = jnp.where(m, k, bidxs[a])

        n = NACC
        while n > 1:
            for a in range(n // 2):
                vl, il = bests[2 * a], bidxs[2 * a]
                vr, ir = bests[2 * a + 1], bidxs[2 * a + 1]
                m = (vr > vl) | ((vr == vl) & (ir < il))
                bests[a] = jnp.where(m, vr, vl)
                bidxs[a] = jnp.where(m, ir, il)
            n //= 2
        best, bidx = bests[0], bidxs[0]

        lanei = lax.broadcasted_iota(jnp.int32, (RB, LANES), 1)
        elem = (bidx << 7) | lanei
        mx = jnp.max(best, axis=1, keepdims=True)
        cand = jnp.where(best == mx, elem, I32_MAX)
        o_ref[pl.ds(i * RB, RB), :] = jnp.min(cand, axis=1, keepdims=True)

        @pl.when(i + NBUF < NBLK)
        def _():
            _blk_copy(x_any, big, sems, i + NBUF, slot).start()

        return 0

    lax.fori_loop(0, NBLK, step, 0)


@jax.jit
def _argmax_tc(x):
    return pl.pallas_call(
        _argmax_body,
        in_specs=[pl.BlockSpec(memory_space=pl.ANY)],
        out_specs=pl.BlockSpec(memory_space=pltpu.VMEM),
        out_shape=jax.ShapeDtypeStruct((ROWS, 1), jnp.int32),
        scratch_shapes=[
            pltpu.VMEM((NBUF * RB, COLS), jnp.float32),
            pltpu.SemaphoreType.DMA((NBUF,)),
        ],
    )(x)


def kernel(inputs):
    return _argmax_tc(inputs)


# speedup vs baseline: 3.9461x; 1.0247x over previous
import jax
import jax.numpy as jnp
from jax import lax
from jax.experimental import pallas as pl
from jax.experimental.pallas import tpu as pltpu

ROWS = 128
COLS = 32768
RB = 8
NBLK = ROWS // RB
LANES = 128
NCH = COLS // LANES
NACC = 8
NBUF = 6
I32_MAX = 2147483647


def _blk_copy(x_any, big, sems, i, slot):
    return pltpu.make_async_copy(
        x_any.at[pl.ds(i * RB, RB)],
        big.at[pl.ds(slot * RB, RB)],
        sems.at[slot])


def _argmax_body(x_any, o_ref, big, sems):
    for i in range(NBUF):
        _blk_copy(x_any, big, sems, i, i).start()

    def step(i, _):
        slot = lax.rem(i, NBUF)
        _blk_copy(x_any, big, sems, i, slot).wait()
        base = slot * RB

        bests = []
        bidxs = []
        for a in range(NACC):
            bests.append(big[pl.ds(base, RB), a * LANES:(a + 1) * LANES])
            bidxs.append(jnp.full((RB, LANES), a, jnp.int32))
        for k in range(NACC, NCH):
            a = k % NACC
            v = big[pl.ds(base, RB), k * LANES:(k + 1) * LANES]
            m = v > bests[a]
            bests[a] = jnp.where(m, v, bests[a])
            bidxs[a] = jnp.where(m, k, bidxs[a])

        n = NACC
        while n > 1:
            for a in range(n // 2):
                vl, il = bests[2 * a], bidxs[2 * a]
                vr, ir = bests[2 * a + 1], bidxs[2 * a + 1]
                m = (vr > vl) | ((vr == vl) & (ir < il))
                bests[a] = jnp.where(m, vr, vl)
                bidxs[a] = jnp.where(m, ir, il)
            n //= 2
        best, bidx = bests[0], bidxs[0]

        lanei = lax.broadcasted_iota(jnp.int32, (RB, LANES), 1)
        elem = (bidx << 7) | lanei
        mx = jnp.max(best, axis=1, keepdims=True)
        cand = jnp.where(best == mx, elem, I32_MAX)
        o_ref[pl.ds(i * RB, RB), :] = jnp.min(cand, axis=1, keepdims=True)

        @pl.when(i + NBUF < NBLK)
        def _():
            _blk_copy(x_any, big, sems, i + NBUF, slot).start()

        return 0

    lax.fori_loop(0, NBLK, step, 0)


@jax.jit
def _argmax_tc(x):
    return pl.pallas_call(
        _argmax_body,
        in_specs=[pl.BlockSpec(memory_space=pl.ANY)],
        out_specs=pl.BlockSpec(memory_space=pltpu.VMEM),
        out_shape=jax.ShapeDtypeStruct((ROWS, 1), jnp.int32),
        scratch_shapes=[
            pltpu.VMEM((NBUF * RB, COLS), jnp.float32),
            pltpu.SemaphoreType.DMA((NBUF,)),
        ],
    )(x)


def kernel(inputs):
    return _argmax_tc(inputs)
